# baseline (speedup 1.0000x reference)
import functools

import jax
import jax.numpy as jnp
from jax import lax
from jax.experimental import pallas as pl
from jax.experimental.pallas import tpu as pltpu
from jax.experimental.pallas import tpu_sc as plsc

NUM_USER = 30000
IDL_BETA = 0.3
NC = 2
NS = 16
LANES = 16
COLS = 64
SUB = 128
EROWS = 3200
E_PAD = EROWS * SUB
STAGE = 8

_MESH = plsc.VectorSubcoreMesh(core_axis_name="c", subcore_axis_name="s")


def _zero_rows(buf, nrows):
    zero = jnp.zeros((LANES,), jnp.float32)

    def body(i, _):
        for q in range(COLS // LANES):
            buf[i, pl.ds(q * LANES, LANES)] = zero
        return 0

    lax.fori_loop(0, nrows, body, 0, unroll=4)


def _spmm_sc(n_acc, mode, yt_off, x_rows):
    rows_per_tile = EROWS // NS
    chunks = rows_per_tile // STAGE
    if n_acc == 30000:
        passes, rpp, acc_rows, stripe, estripe = 2, 15000, 15040, 940, 188
    else:
        passes, rpp, acc_rows, stripe, estripe = 1, n_acc, n_acc, n_acc // NS, n_acc // NS // 5
    n_ep = stripe // estripe
    n_out = (passes - 1) * rpp + acc_rows

    scratch = [
        pltpu.VMEM_SHARED((acc_rows, COLS), jnp.float32),
        pltpu.VMEM((STAGE, SUB), jnp.int32),
        pltpu.VMEM((STAGE, SUB), jnp.int32),
        pltpu.VMEM((STAGE, SUB), jnp.int32),
        pltpu.VMEM((STAGE, SUB), jnp.float32),
        pltpu.VMEM((STAGE, SUB), jnp.float32),
        pltpu.VMEM((SUB, COLS), jnp.float32),
        pltpu.VMEM((SUB, COLS), jnp.float32),
        pltpu.VMEM((estripe, COLS), jnp.float32),
        pltpu.VMEM((estripe, COLS), jnp.float32),
        pltpu.SemaphoreType.DMA,
        pltpu.SemaphoreType.DMA,
    ]

    def body(*refs):
        if mode == 0:
            (x_hbm, gi_hbm, si_hbm, ev_hbm, out_hbm,
             acc, gi_v, si_v, sl_v, ev_v, em_v, gb0, gb1, obuf, ytbuf,
             gsem0, gsem1) = refs
            yt_hbm = None
        else:
            (x_hbm, gi_hbm, si_hbm, ev_hbm, yt_hbm, out_hbm,
             acc, gi_v, si_v, sl_v, ev_v, em_v, gb0, gb1, obuf, ytbuf,
             gsem0, gsem1) = refs
        c = lax.axis_index("c")
        s = lax.axis_index("s")

        gbufs = (gb0, gb1)
        gsems = (gsem0, gsem1)

        def scale(gbuf, vref, j):
            def sgrp(g, _):
                v16 = vref[j, pl.ds(g * LANES, LANES)]
                base = g * LANES
                for l in range(LANES):
                    v = v16[l]
                    for q in range(COLS // LANES):
                        sl = pl.ds(q * LANES, LANES)
                        gbuf[base + l, sl] = gbuf[base + l, sl] * v
                return 0

            lax.fori_loop(0, SUB // LANES, sgrp, 0)

        erow0 = s * rows_per_tile

        for p in range(passes):
            _zero_rows(obuf, estripe)
            for t in range(n_ep):
                pltpu.sync_copy(obuf, acc.at[pl.ds(s * stripe + t * estripe,
                                                   estripe)])
            plsc.subcore_barrier()

            def chunk(t, _):
                r0 = erow0 + t * STAGE
                pltpu.sync_copy(gi_hbm.at[pl.ds(r0, STAGE)], gi_v)
                pltpu.sync_copy(si_hbm.at[pl.ds(r0, STAGE)], si_v)
                pltpu.sync_copy(ev_hbm.at[pl.ds(r0, STAGE)], ev_v)
                if passes > 1:
                    lo = p * rpp
                    def xform(j, _):
                        for g in range(SUB // LANES):
                            sl = pl.ds(g * LANES, LANES)
                            si16 = si_v[j, sl]
                            v16 = ev_v[j, sl]
                            owned = (si16 >= lo) & (si16 < lo + rpp)
                            sl_v[j, sl] = jnp.where(
                                owned, si16 - lo, si16 & 8191)
                            em_v[j, sl] = jnp.where(owned, v16, 0.0)
                        return 0
                    lax.fori_loop(0, STAGE, xform, 0)
                    sref, vref = sl_v, em_v
                else:
                    sref, vref = si_v, ev_v
                cps = [None] * STAGE
                cps[0] = pltpu.async_copy(
                    x_hbm.at[c].at[gi_v.at[0]], gbufs[0], gsems[0])
                for j in range(STAGE):
                    cps[j].wait()
                    if j + 1 < STAGE:
                        cps[j + 1] = pltpu.async_copy(
                            x_hbm.at[c].at[gi_v.at[j + 1]],
                            gbufs[(j + 1) % 2], gsems[(j + 1) % 2])
                    gbuf = gbufs[j % 2]
                    scale(gbuf, vref, j)
                    pltpu.sync_copy(gbuf, acc.at[sref.at[j]], add=True)
                return 0

            lax.fori_loop(0, chunks, chunk, 0)
            plsc.subcore_barrier()

            for t in range(n_ep):
                base = s * stripe + t * estripe
                gbase = p * rpp + base
                if mode == 0:
                    pltpu.sync_copy(acc.at[pl.ds(base, estripe)],
                                    out_hbm.at[c].at[pl.ds(gbase, estripe)])
                else:
                    pltpu.sync_copy(acc.at[pl.ds(base, estripe)], obuf)
                    pltpu.sync_copy(
                        yt_hbm.at[c].at[pl.ds(yt_off + gbase, estripe)],
                        ytbuf)

                    def crow(k, _):
                        for q in range(COLS // LANES):
                            sl = pl.ds(q * LANES, LANES)
                            a = obuf[k, sl]
                            bv = ytbuf[k, sl]
                            obuf[k, sl] = bv + a if mode == 1 else bv - a
                        return 0

                    lax.fori_loop(0, estripe, crow, 0, unroll=4)
                    pltpu.sync_copy(obuf,
                                    out_hbm.at[c].at[pl.ds(gbase, estripe)])
            if p + 1 < passes:
                plsc.subcore_barrier()

    fn = pl.kernel(
        body,
        out_type=jax.ShapeDtypeStruct((NC, n_out, COLS), jnp.float32),
        mesh=_MESH,
        scratch_types=scratch,
        compiler_params=pltpu.CompilerParams(use_tc_tiling_on_sc=False),
    )
    return lambda *a: fn(*a)[:, :n_acc]


AT_STRIPE = 40960
AT_PAD = NS * AT_STRIPE
POS_PAD = 30720


def _build_at_kernel(i_n):
    trash = i_n * COLS
    rows_per_tile = EROWS // NS
    chunks = rows_per_tile // STAGE

    scratch = [
        pltpu.VMEM((8192,), jnp.float32),
        pltpu.VMEM((STAGE, SUB), jnp.int32),
        pltpu.VMEM((STAGE, SUB), jnp.int32),
        pltpu.VMEM((STAGE, SUB), jnp.float32),
        pltpu.VMEM((SUB,), jnp.int32),
        pltpu.VMEM((SUB,), jnp.int32),
    ]

    def body(pos_hbm, eu_hbm, ei_hbm, ev_hbm, out_hbm,
             zf, eu_v, ei_v, ev_v, p_v, sidx):
        c = lax.axis_index("c")
        s = lax.axis_index("s")

        zero = jnp.zeros((LANES,), jnp.float32)

        def zrow(i, _):
            zf[pl.ds(i * LANES, LANES)] = zero
            return 0

        lax.fori_loop(0, 8192 // LANES, zrow, 0, unroll=4)
        for t in range(AT_STRIPE // 8192):
            pltpu.sync_copy(
                zf, out_hbm.at[c].at[pl.ds(s * AT_STRIPE + t * 8192, 8192)])
        plsc.subcore_barrier()

        erow0 = s * rows_per_tile

        def chunk(t, _):
            r0 = erow0 + t * STAGE
            pltpu.sync_copy(eu_hbm.at[pl.ds(r0, STAGE)], eu_v)
            pltpu.sync_copy(ei_hbm.at[pl.ds(r0, STAGE)], ei_v)
            pltpu.sync_copy(ev_hbm.at[pl.ds(r0, STAGE)], ev_v)
            for j in range(STAGE):
                pltpu.sync_copy(pos_hbm.at[eu_v.at[j]], p_v)
                for g in range(SUB // LANES):
                    sl = pl.ds(g * LANES, LANES)
                    ei16 = ei_v[j, sl]
                    p16 = p_v[sl]
                    ok = (p16 < SUB) & ((p16 >> 6) == c)
                    flat = jnp.where(ok, ei16 * COLS + (p16 & (COLS - 1)),
                                     trash)
                    sidx[sl] = flat
                pltpu.sync_copy(ev_v.at[j], out_hbm.at[c].at[sidx])
            return 0

        lax.fori_loop(0, chunks, chunk, 0)

    return pl.kernel(
        body,
        out_type=jax.ShapeDtypeStruct((NC, AT_PAD), jnp.float32),
        mesh=_MESH,
        scratch_types=scratch,
        compiler_params=pltpu.CompilerParams(use_tc_tiling_on_sc=False),
    )


def _idl_c_kernel(left_ref, at_ref, c_ref):
    @pl.when(pl.program_id(0) == 0)
    def _():
        c_ref[...] = jnp.zeros_like(c_ref)

    c_ref[...] += lax.dot_general(
        left_ref[...], at_ref[...], (((0,), (0,)), ((), ())),
        preferred_element_type=jnp.float32)

    @pl.when(pl.program_id(0) == pl.num_programs(0) - 1)
    def _():
        c_ref[...] = c_ref[...] * IDL_BETA


def _idl_out_kernel(right_ref, c_ref, out_ref):
    out_ref[...] = lax.dot_general(
        right_ref[...], c_ref[...], (((0,), (0,)), ((), ())),
        preferred_element_type=jnp.float32)


def _idl_t(left_i, right_mat, at_b):
    ii, f = left_i.shape
    n = right_mat.shape[1]
    b = at_b.shape[1]
    ti = 2000
    c = pl.pallas_call(
        _idl_c_kernel,
        grid=(ii // ti,),
        in_specs=[pl.BlockSpec((ti, f), lambda i: (i, 0)),
                  pl.BlockSpec((ti, b), lambda i: (i, 0))],
        out_specs=pl.BlockSpec((f, b), lambda i: (0, 0)),
        out_shape=jax.ShapeDtypeStruct((f, b), jnp.float32),
    )(left_i, at_b)
    tn = 4096
    return pl.pallas_call(
        _idl_out_kernel,
        grid=(pl.cdiv(n, tn),),
        in_specs=[pl.BlockSpec((f, tn), lambda i: (0, i)),
                  pl.BlockSpec((f, b), lambda i: (0, 0))],
        out_specs=pl.BlockSpec((tn, b), lambda i: (i, 0)),
        out_shape=jax.ShapeDtypeStruct((n, b), jnp.float32),
    )(right_mat, c)


def kernel(batch_users, edge_rows, edge_cols, edge_vals, left_mat, right_mat):
    e2 = edge_rows.shape[0]
    e = e2 // 2
    n, f = left_mat.shape
    u_n = NUM_USER
    i_n = n - u_n
    b = batch_users.shape[0]
    pad = E_PAD - e

    eu = edge_rows[:e]
    ei = edge_cols[:e] - u_n
    ev = edge_vals[:e]
    zi = jnp.zeros((pad,), jnp.int32)
    eu_s = jnp.concatenate([eu, zi]).reshape(EROWS, SUB)
    eu_b = jnp.concatenate([eu, jnp.full((pad,), u_n, jnp.int32)]
                           ).reshape(EROWS, SUB)
    ei_s = jnp.concatenate([ei, zi]).reshape(EROWS, SUB)
    ev_p = jnp.concatenate([ev, jnp.zeros((pad,), jnp.float32)]
                           ).reshape(EROWS, SUB)

    pos_tab = jnp.full((POS_PAD,), b, jnp.int32).at[batch_users].set(
        jnp.arange(b, dtype=jnp.int32))

    at_flat = _build_at_kernel(i_n)(pos_tab, eu_b, ei_s, ev_p)
    xs_at = at_flat[:, :i_n * COLS].reshape(NC, i_n, COLS)
    at_b = xs_at.transpose(1, 0, 2).reshape(i_n, b)

    yt0 = _idl_t(left_mat[u_n:], right_mat, at_b)
    yt0_s = yt0.reshape(n, NC, COLS).transpose(1, 0, 2)

    spmm_ui = _spmm_sc(u_n, 0, 0, i_n)
    s1 = spmm_ui(xs_at, ei_s, eu_s, ev_p)
    yt_i = _spmm_sc(i_n, 1, u_n, u_n)(s1, eu_s, ei_s, ev_p, yt0_s)
    q1 = _spmm_sc(i_n, 0, 0, n)(yt0_s, eu_s, ei_s, ev_p)
    out_u = _spmm_sc(u_n, 2, 0, i_n)(q1, ei_s, eu_s, ev_p, yt0_s)
    r1 = spmm_ui(yt_i, ei_s, eu_s, ev_p)
    out_i = _spmm_sc(i_n, 2, 0, u_n)(r1, eu_s, ei_s, ev_p, yt_i)

    out_u_b = out_u.transpose(0, 2, 1).reshape(b, u_n)
    out_i_b = out_i.transpose(0, 2, 1).reshape(b, i_n)
    return jnp.concatenate([out_u_b, out_i_b], axis=1)


# speedup vs baseline: 7.6782x; 7.6782x over previous
import functools

import jax
import jax.numpy as jnp
from jax import lax
from jax.experimental import pallas as pl
from jax.experimental.pallas import tpu as pltpu
from jax.experimental.pallas import tpu_sc as plsc

NUM_USER = 30000
IDL_BETA = 0.3
NC = 2
NS = 16
LANES = 16
COLS = 64
SUB = 128
EROWS = 3200
E_PAD = EROWS * SUB
STAGE = 8

_MESH = plsc.VectorSubcoreMesh(core_axis_name="c", subcore_axis_name="s")


def _zero_rows(buf, nrows):
    zero = jnp.zeros((LANES,), jnp.float32)

    def body(i, _):
        for q in range(COLS // LANES):
            buf[i, pl.ds(q * LANES, LANES)] = zero
        return 0

    lax.fori_loop(0, nrows, body, 0, unroll=4)


def _spmm_sc(n_acc, mode, yt_off, x_rows):
    rows_per_tile = EROWS // NS
    chunks = rows_per_tile // STAGE
    if n_acc == 30000:
        passes, rpp, acc_rows, stripe, estripe = 2, 15000, 15040, 940, 188
    else:
        passes, rpp, acc_rows, stripe, estripe = 1, n_acc, n_acc, n_acc // NS, n_acc // NS // 5
    n_ep = stripe // estripe
    n_out = (passes - 1) * rpp + acc_rows

    scratch = [
        pltpu.VMEM_SHARED((acc_rows, COLS), jnp.float32),
    ] + ([pltpu.VMEM_SHARED((acc_rows, COLS), jnp.float32)]
         if mode == 3 else []) + [
        pltpu.VMEM((STAGE, SUB), jnp.int32),
        pltpu.VMEM((STAGE, SUB), jnp.int32),
        pltpu.VMEM((STAGE, SUB), jnp.int32),
        pltpu.VMEM((STAGE, SUB), jnp.float32),
        pltpu.VMEM((STAGE, SUB), jnp.float32),
        pltpu.VMEM((SUB, COLS), jnp.float32),
        pltpu.VMEM((SUB, COLS), jnp.float32),
        pltpu.VMEM((estripe, COLS), jnp.float32),
        pltpu.VMEM((estripe, COLS), jnp.float32),
        pltpu.SemaphoreType.DMA,
        pltpu.SemaphoreType.DMA,
    ]

    def body(*refs):
        acc2 = None
        if mode == 0:
            (x_hbm, gi_hbm, si_hbm, ev_hbm, out_hbm,
             acc, gi_v, si_v, sl_v, ev_v, em_v, gb0, gb1, obuf, ytbuf,
             gsem0, gsem1) = refs
            yt_hbm = None
        elif mode == 3:
            (x_hbm, gi_hbm, si_hbm, ev_hbm, out_hbm,
             acc, acc2, gi_v, si_v, sl_v, ev_v, em_v, gb0, gb1, obuf, ytbuf,
             gsem0, gsem1) = refs
            yt_hbm = None
        else:
            (x_hbm, gi_hbm, si_hbm, ev_hbm, yt_hbm, out_hbm,
             acc, gi_v, si_v, sl_v, ev_v, em_v, gb0, gb1, obuf, ytbuf,
             gsem0, gsem1) = refs
        c = lax.axis_index("c")
        s = lax.axis_index("s")

        gbufs = (gb0, gb1)
        gsems = (gsem0, gsem1)

        def scale(gbuf, vref, j):
            def sgrp(g, _):
                v16 = vref[j, pl.ds(g * LANES, LANES)]
                base = g * LANES
                for l in range(LANES):
                    v = v16[l]
                    for q in range(COLS // LANES):
                        sl = pl.ds(q * LANES, LANES)
                        gbuf[base + l, sl] = gbuf[base + l, sl] * v
                return 0

            lax.fori_loop(0, SUB // LANES, sgrp, 0)

        erow0 = s * rows_per_tile

        for p in range(passes):
            _zero_rows(obuf, estripe)
            for t in range(n_ep):
                pltpu.sync_copy(obuf, acc.at[pl.ds(s * stripe + t * estripe,
                                                   estripe)])
                if mode == 3:
                    pltpu.sync_copy(
                        obuf, acc2.at[pl.ds(s * stripe + t * estripe,
                                            estripe)])
            plsc.subcore_barrier()

            def chunk(t, _):
                r0 = erow0 + t * STAGE
                pltpu.sync_copy(gi_hbm.at[pl.ds(r0, STAGE)], gi_v)
                pltpu.sync_copy(si_hbm.at[pl.ds(r0, STAGE)], si_v)
                pltpu.sync_copy(ev_hbm.at[pl.ds(r0, STAGE)], ev_v)
                if passes > 1:
                    lo = p * rpp
                    def xform(j, _):
                        for g in range(SUB // LANES):
                            sl = pl.ds(g * LANES, LANES)
                            si16 = si_v[j, sl]
                            v16 = ev_v[j, sl]
                            owned = (si16 >= lo) & (si16 < lo + rpp)
                            sl_v[j, sl] = jnp.where(
                                owned, si16 - lo, si16 & 8191)
                            em_v[j, sl] = jnp.where(owned, v16, 0.0)
                        return 0
                    lax.fori_loop(0, STAGE, xform, 0)
                    sref, vref = sl_v, em_v
                else:
                    sref, vref = si_v, ev_v
                cps = [None] * STAGE
                cps[0] = pltpu.async_copy(
                    x_hbm.at[c].at[gi_v.at[0]], gbufs[0], gsems[0])
                for j in range(STAGE):
                    cps[j].wait()
                    if j + 1 < STAGE:
                        cps[j + 1] = pltpu.async_copy(
                            x_hbm.at[c].at[gi_v.at[j + 1]],
                            gbufs[(j + 1) % 2], gsems[(j + 1) % 2])
                    gbuf = gbufs[j % 2]
                    if mode == 3:
                        pltpu.sync_copy(gbuf, acc2.at[sref.at[j]], add=True)
                    scale(gbuf, vref, j)
                    pltpu.sync_copy(gbuf, acc.at[sref.at[j]], add=True)
                return 0

            lax.fori_loop(0, chunks, chunk, 0)
            plsc.subcore_barrier()

            for t in range(n_ep):
                base = s * stripe + t * estripe
                gbase = p * rpp + base
                if mode == 0:
                    pltpu.sync_copy(acc.at[pl.ds(base, estripe)],
                                    out_hbm.at[c].at[pl.ds(gbase, estripe)])
                elif mode == 3:
                    pltpu.sync_copy(acc.at[pl.ds(base, estripe)], obuf)
                    pltpu.sync_copy(acc2.at[pl.ds(base, estripe)], ytbuf)

                    def drow(k, _):
                        for q in range(COLS // LANES):
                            sl = pl.ds(q * LANES, LANES)
                            cnt = jnp.maximum(ytbuf[k, sl], 1.0)
                            obuf[k, sl] = obuf[k, sl] / cnt
                        return 0

                    lax.fori_loop(0, estripe, drow, 0, unroll=4)
                    pltpu.sync_copy(obuf,
                                    out_hbm.at[c].at[pl.ds(gbase, estripe)])
                else:
                    pltpu.sync_copy(acc.at[pl.ds(base, estripe)], obuf)
                    pltpu.sync_copy(
                        yt_hbm.at[c].at[pl.ds(yt_off + gbase, estripe)],
                        ytbuf)

                    def crow(k, _):
                        for q in range(COLS // LANES):
                            sl = pl.ds(q * LANES, LANES)
                            a = obuf[k, sl]
                            bv = ytbuf[k, sl]
                            obuf[k, sl] = bv + a if mode == 1 else bv - a
                        return 0

                    lax.fori_loop(0, estripe, crow, 0, unroll=4)
                    pltpu.sync_copy(obuf,
                                    out_hbm.at[c].at[pl.ds(gbase, estripe)])
            if p + 1 < passes:
                plsc.subcore_barrier()

    fn = pl.kernel(
        body,
        out_type=jax.ShapeDtypeStruct((NC, n_out, COLS), jnp.float32),
        mesh=_MESH,
        scratch_types=scratch,
        compiler_params=pltpu.CompilerParams(use_tc_tiling_on_sc=False),
    )
    return lambda *a: fn(*a)[:, :n_acc]


def _idl_c_kernel(left_ref, at_ref, c_ref):
    @pl.when(pl.program_id(0) == 0)
    def _():
        c_ref[...] = jnp.zeros_like(c_ref)

    c_ref[...] += lax.dot_general(
        left_ref[...], at_ref[...], (((0,), (0,)), ((), ())),
        preferred_element_type=jnp.float32)

    @pl.when(pl.program_id(0) == pl.num_programs(0) - 1)
    def _():
        c_ref[...] = c_ref[...] * IDL_BETA


def _idl_out_kernel(right_ref, c_ref, out_ref):
    out_ref[...] = lax.dot_general(
        right_ref[...], c_ref[...], (((0,), (0,)), ((), ())),
        preferred_element_type=jnp.float32)


def _idl_t(left_i, right_mat, at_b):
    ii, f = left_i.shape
    n = right_mat.shape[1]
    b = at_b.shape[1]
    ti = 2000
    c = pl.pallas_call(
        _idl_c_kernel,
        grid=(ii // ti,),
        in_specs=[pl.BlockSpec((ti, f), lambda i: (i, 0)),
                  pl.BlockSpec((ti, b), lambda i: (i, 0))],
        out_specs=pl.BlockSpec((f, b), lambda i: (0, 0)),
        out_shape=jax.ShapeDtypeStruct((f, b), jnp.float32),
    )(left_i, at_b)
    tn = 4096
    return pl.pallas_call(
        _idl_out_kernel,
        grid=(pl.cdiv(n, tn),),
        in_specs=[pl.BlockSpec((f, tn), lambda i: (0, i)),
                  pl.BlockSpec((f, b), lambda i: (0, 0))],
        out_specs=pl.BlockSpec((tn, b), lambda i: (i, 0)),
        out_shape=jax.ShapeDtypeStruct((n, b), jnp.float32),
    )(right_mat, c)


def kernel(batch_users, edge_rows, edge_cols, edge_vals, left_mat, right_mat):
    e2 = edge_rows.shape[0]
    e = e2 // 2
    n, f = left_mat.shape
    u_n = NUM_USER
    i_n = n - u_n
    b = batch_users.shape[0]
    pad = E_PAD - e

    eu = edge_rows[:e]
    ei = edge_cols[:e] - u_n
    ev = edge_vals[:e]
    zi = jnp.zeros((pad,), jnp.int32)
    eu_s = jnp.concatenate([eu, zi]).reshape(EROWS, SUB)
    ei_s = jnp.concatenate([ei, zi]).reshape(EROWS, SUB)
    ev_p = jnp.concatenate([ev, jnp.zeros((pad,), jnp.float32)]
                           ).reshape(EROWS, SUB)

    pos_tab = jnp.full((u_n,), b, jnp.int32).at[batch_users].set(
        jnp.arange(b, dtype=jnp.int32))
    p_p = jnp.concatenate([pos_tab[eu], jnp.full((pad,), b, jnp.int32)]
                          ).reshape(EROWS, SUB)
    x_id = jnp.concatenate(
        [jnp.eye(b, dtype=jnp.float32).reshape(b, NC, COLS).transpose(
            1, 0, 2), jnp.zeros((NC, 8, COLS), jnp.float32)], axis=1)

    xs_at = _spmm_sc(i_n, 3, 0, b)(x_id, p_p, ei_s, ev_p)
    at_b = xs_at.transpose(1, 0, 2).reshape(i_n, b)

    yt0 = _idl_t(left_mat[u_n:], right_mat, at_b)
    yt0_s = yt0.reshape(n, NC, COLS).transpose(1, 0, 2)

    spmm_ui = _spmm_sc(u_n, 0, 0, i_n)
    s1 = spmm_ui(xs_at, ei_s, eu_s, ev_p)
    yt_i = _spmm_sc(i_n, 1, u_n, u_n)(s1, eu_s, ei_s, ev_p, yt0_s)
    q1 = _spmm_sc(i_n, 0, 0, n)(yt0_s, eu_s, ei_s, ev_p)
    out_u = _spmm_sc(u_n, 2, 0, i_n)(q1, ei_s, eu_s, ev_p, yt0_s)
    r1 = spmm_ui(yt_i, ei_s, eu_s, ev_p)
    out_i = _spmm_sc(i_n, 2, 0, u_n)(r1, eu_s, ei_s, ev_p, yt_i)

    out_u_b = out_u.transpose(0, 2, 1).reshape(b, u_n)
    out_i_b = out_i.transpose(0, 2, 1).reshape(b, i_n)
    return jnp.concatenate([out_u_b, out_i_b], axis=1)


# speedup vs baseline: 7.9389x; 1.0340x over previous
import functools

import jax
import jax.numpy as jnp
from jax import lax
from jax.experimental import pallas as pl
from jax.experimental.pallas import tpu as pltpu
from jax.experimental.pallas import tpu_sc as plsc

NUM_USER = 30000
IDL_BETA = 0.3
NC = 2
NS = 16
LANES = 16
COLS = 64
SUB = 128
EROWS = 3200
E_PAD = EROWS * SUB
STAGE = 8

_MESH = plsc.VectorSubcoreMesh(core_axis_name="c", subcore_axis_name="s")


def _zero_rows(buf, nrows):
    zero = jnp.zeros((LANES,), jnp.float32)

    def body(i, _):
        for q in range(COLS // LANES):
            buf[i, pl.ds(q * LANES, LANES)] = zero
        return 0

    lax.fori_loop(0, nrows, body, 0, unroll=4)


def _spmm_sc(n_acc, mode, yt_off, x_rows):
    rows_per_tile = EROWS // NS
    chunks = rows_per_tile // STAGE
    if n_acc == 30000:
        passes, rpp, acc_rows, stripe, estripe = 2, 15000, 15040, 940, 188
    else:
        passes, rpp, acc_rows, stripe, estripe = 1, n_acc, n_acc, n_acc // NS, n_acc // NS // 5
    n_ep = stripe // estripe
    n_out = (passes - 1) * rpp + acc_rows

    scratch = [
        pltpu.VMEM_SHARED((acc_rows, COLS), jnp.float32),
    ] + ([pltpu.VMEM_SHARED((acc_rows, COLS), jnp.float32)]
         if mode == 3 else []) + [
        pltpu.VMEM((STAGE, SUB), jnp.int32),
        pltpu.VMEM((STAGE, SUB), jnp.int32),
        pltpu.VMEM((STAGE, SUB), jnp.int32),
        pltpu.VMEM((STAGE, SUB), jnp.float32),
        pltpu.VMEM((STAGE, SUB), jnp.float32),
    ] + [pltpu.VMEM((SUB, COLS), jnp.float32)] * (2 if mode == 3 else 4) + [
        pltpu.VMEM((estripe, COLS), jnp.float32),
        pltpu.VMEM((estripe, COLS), jnp.float32),
    ] + [pltpu.SemaphoreType.DMA] * 8

    def body(*refs):
        acc2 = None
        if mode == 0:
            (x_hbm, gi_hbm, si_hbm, ev_hbm, out_hbm,
             acc, gi_v, si_v, sl_v, ev_v, em_v, gb0, gb1, gb2, gb3,
             obuf, ytbuf, *sems) = refs
            yt_hbm = None
        elif mode == 3:
            (x_hbm, gi_hbm, si_hbm, ev_hbm, out_hbm,
             acc, acc2, gi_v, si_v, sl_v, ev_v, em_v, gb0, gb1,
             obuf, ytbuf, *sems) = refs
            gb2 = gb3 = None
            yt_hbm = None
        else:
            (x_hbm, gi_hbm, si_hbm, ev_hbm, yt_hbm, out_hbm,
             acc, gi_v, si_v, sl_v, ev_v, em_v, gb0, gb1, gb2, gb3,
             obuf, ytbuf, *sems) = refs
        c = lax.axis_index("c")
        s = lax.axis_index("s")

        ring = 2 if mode == 3 else 4
        gbufs = (gb0, gb1, gb2, gb3)[:ring]
        gsems = sems[:4]
        ssems = sems[4:]

        def scale(gbuf, vref, j):
            def sgrp(g, _):
                v16 = vref[j, pl.ds(g * LANES, LANES)]
                base = g * LANES
                for l in range(LANES):
                    v = v16[l]
                    for q in range(COLS // LANES):
                        sl = pl.ds(q * LANES, LANES)
                        gbuf[base + l, sl] = gbuf[base + l, sl] * v
                return 0

            lax.fori_loop(0, SUB // LANES, sgrp, 0)

        erow0 = s * rows_per_tile

        for p in range(passes):
            _zero_rows(obuf, estripe)
            for t in range(n_ep):
                pltpu.sync_copy(obuf, acc.at[pl.ds(s * stripe + t * estripe,
                                                   estripe)])
                if mode == 3:
                    pltpu.sync_copy(
                        obuf, acc2.at[pl.ds(s * stripe + t * estripe,
                                            estripe)])
            plsc.subcore_barrier()

            def chunk(t, _):
                r0 = erow0 + t * STAGE
                pltpu.sync_copy(gi_hbm.at[pl.ds(r0, STAGE)], gi_v)
                pltpu.sync_copy(si_hbm.at[pl.ds(r0, STAGE)], si_v)
                pltpu.sync_copy(ev_hbm.at[pl.ds(r0, STAGE)], ev_v)
                if passes > 1:
                    lo = p * rpp
                    def xform(j, _):
                        for g in range(SUB // LANES):
                            sl = pl.ds(g * LANES, LANES)
                            si16 = si_v[j, sl]
                            v16 = ev_v[j, sl]
                            owned = (si16 >= lo) & (si16 < lo + rpp)
                            sl_v[j, sl] = jnp.where(
                                owned, si16 - lo, si16 & 8191)
                            em_v[j, sl] = jnp.where(owned, v16, 0.0)
                        return 0
                    lax.fori_loop(0, STAGE, xform, 0)
                    sref, vref = sl_v, em_v
                else:
                    sref, vref = si_v, ev_v
                ahead = ring // 2
                gcp = [None] * STAGE
                scp = [None] * STAGE
                for j in range(ahead):
                    gcp[j] = pltpu.async_copy(
                        x_hbm.at[c].at[gi_v.at[j]], gbufs[j], gsems[j])
                for j in range(STAGE):
                    gcp[j].wait()
                    if j + ahead < STAGE:
                        if j + ahead - ring >= 0:
                            scp[j + ahead - ring].wait()
                        r = (j + ahead) % ring
                        gcp[j + ahead] = pltpu.async_copy(
                            x_hbm.at[c].at[gi_v.at[j + ahead]],
                            gbufs[r], gsems[r])
                    gbuf = gbufs[j % ring]
                    if mode == 3:
                        pltpu.sync_copy(gbuf, acc2.at[sref.at[j]], add=True)
                    scale(gbuf, vref, j)
                    scp[j] = pltpu.async_copy(
                        gbuf, acc.at[sref.at[j]], ssems[j % ring], add=True)
                for j in range(STAGE - ring + (STAGE - ring < 0 and STAGE or 0), STAGE):
                    scp[j].wait()
                return 0

            lax.fori_loop(0, chunks, chunk, 0)
            plsc.subcore_barrier()

            for t in range(n_ep):
                base = s * stripe + t * estripe
                gbase = p * rpp + base
                if mode == 0:
                    pltpu.sync_copy(acc.at[pl.ds(base, estripe)],
                                    out_hbm.at[c].at[pl.ds(gbase, estripe)])
                elif mode == 3:
                    pltpu.sync_copy(acc.at[pl.ds(base, estripe)], obuf)
                    pltpu.sync_copy(acc2.at[pl.ds(base, estripe)], ytbuf)

                    def drow(k, _):
                        for q in range(COLS // LANES):
                            sl = pl.ds(q * LANES, LANES)
                            cnt = jnp.maximum(ytbuf[k, sl], 1.0)
                            obuf[k, sl] = obuf[k, sl] / cnt
                        return 0

                    lax.fori_loop(0, estripe, drow, 0, unroll=4)
                    pltpu.sync_copy(obuf,
                                    out_hbm.at[c].at[pl.ds(gbase, estripe)])
                else:
                    pltpu.sync_copy(acc.at[pl.ds(base, estripe)], obuf)
                    pltpu.sync_copy(
                        yt_hbm.at[c].at[pl.ds(yt_off + gbase, estripe)],
                        ytbuf)

                    def crow(k, _):
                        for q in range(COLS // LANES):
                            sl = pl.ds(q * LANES, LANES)
                            a = obuf[k, sl]
                            bv = ytbuf[k, sl]
                            obuf[k, sl] = bv + a if mode == 1 else bv - a
                        return 0

                    lax.fori_loop(0, estripe, crow, 0, unroll=4)
                    pltpu.sync_copy(obuf,
                                    out_hbm.at[c].at[pl.ds(gbase, estripe)])
            if p + 1 < passes:
                plsc.subcore_barrier()

    fn = pl.kernel(
        body,
        out_type=jax.ShapeDtypeStruct((NC, n_out, COLS), jnp.float32),
        mesh=_MESH,
        scratch_types=scratch,
        compiler_params=pltpu.CompilerParams(use_tc_tiling_on_sc=False),
    )
    return lambda *a: fn(*a)[:, :n_acc]


def _idl_c_kernel(left_ref, at_ref, c_ref):
    @pl.when(pl.program_id(0) == 0)
    def _():
        c_ref[...] = jnp.zeros_like(c_ref)

    c_ref[...] += lax.dot_general(
        left_ref[...], at_ref[...], (((0,), (0,)), ((), ())),
        preferred_element_type=jnp.float32)

    @pl.when(pl.program_id(0) == pl.num_programs(0) - 1)
    def _():
        c_ref[...] = c_ref[...] * IDL_BETA


def _idl_out_kernel(right_ref, c_ref, out_ref):
    out_ref[...] = lax.dot_general(
        right_ref[...], c_ref[...], (((0,), (0,)), ((), ())),
        preferred_element_type=jnp.float32)


def _idl_t(left_i, right_mat, at_b):
    ii, f = left_i.shape
    n = right_mat.shape[1]
    b = at_b.shape[1]
    ti = 2000
    c = pl.pallas_call(
        _idl_c_kernel,
        grid=(ii // ti,),
        in_specs=[pl.BlockSpec((ti, f), lambda i: (i, 0)),
                  pl.BlockSpec((ti, b), lambda i: (i, 0))],
        out_specs=pl.BlockSpec((f, b), lambda i: (0, 0)),
        out_shape=jax.ShapeDtypeStruct((f, b), jnp.float32),
    )(left_i, at_b)
    tn = 4096
    return pl.pallas_call(
        _idl_out_kernel,
        grid=(pl.cdiv(n, tn),),
        in_specs=[pl.BlockSpec((f, tn), lambda i: (0, i)),
                  pl.BlockSpec((f, b), lambda i: (0, 0))],
        out_specs=pl.BlockSpec((tn, b), lambda i: (i, 0)),
        out_shape=jax.ShapeDtypeStruct((n, b), jnp.float32),
    )(right_mat, c)


def kernel(batch_users, edge_rows, edge_cols, edge_vals, left_mat, right_mat):
    e2 = edge_rows.shape[0]
    e = e2 // 2
    n, f = left_mat.shape
    u_n = NUM_USER
    i_n = n - u_n
    b = batch_users.shape[0]
    pad = E_PAD - e

    eu = edge_rows[:e]
    ei = edge_cols[:e] - u_n
    ev = edge_vals[:e]
    zi = jnp.zeros((pad,), jnp.int32)
    eu_s = jnp.concatenate([eu, zi]).reshape(EROWS, SUB)
    ei_s = jnp.concatenate([ei, zi]).reshape(EROWS, SUB)
    ev_p = jnp.concatenate([ev, jnp.zeros((pad,), jnp.float32)]
                           ).reshape(EROWS, SUB)

    pos_tab = jnp.full((u_n,), b, jnp.int32).at[batch_users].set(
        jnp.arange(b, dtype=jnp.int32))
    p_p = jnp.concatenate([pos_tab[eu], jnp.full((pad,), b, jnp.int32)]
                          ).reshape(EROWS, SUB)
    x_id = jnp.concatenate(
        [jnp.eye(b, dtype=jnp.float32).reshape(b, NC, COLS).transpose(
            1, 0, 2), jnp.zeros((NC, 8, COLS), jnp.float32)], axis=1)

    xs_at = _spmm_sc(i_n, 3, 0, b)(x_id, p_p, ei_s, ev_p)
    at_b = xs_at.transpose(1, 0, 2).reshape(i_n, b)

    yt0 = _idl_t(left_mat[u_n:], right_mat, at_b)
    yt0_s = yt0.reshape(n, NC, COLS).transpose(1, 0, 2)

    spmm_ui = _spmm_sc(u_n, 0, 0, i_n)
    s1 = spmm_ui(xs_at, ei_s, eu_s, ev_p)
    yt_i = _spmm_sc(i_n, 1, u_n, u_n)(s1, eu_s, ei_s, ev_p, yt0_s)
    q1 = _spmm_sc(i_n, 0, 0, n)(yt0_s, eu_s, ei_s, ev_p)
    out_u = _spmm_sc(u_n, 2, 0, i_n)(q1, ei_s, eu_s, ev_p, yt0_s)
    r1 = spmm_ui(yt_i, ei_s, eu_s, ev_p)
    out_i = _spmm_sc(i_n, 2, 0, u_n)(r1, eu_s, ei_s, ev_p, yt_i)

    out_u_b = out_u.transpose(0, 2, 1).reshape(b, u_n)
    out_i_b = out_i.transpose(0, 2, 1).reshape(b, i_n)
    return jnp.concatenate([out_u_b, out_i_b], axis=1)


# speedup vs baseline: 13.1694x; 1.6588x over previous
import functools

import jax
import jax.numpy as jnp
from jax import lax
from jax.experimental import pallas as pl
from jax.experimental.pallas import tpu as pltpu
from jax.experimental.pallas import tpu_sc as plsc

NUM_USER = 30000
IDL_BETA = 0.3
NC = 2
NS = 16
LANES = 16
COLS = 64
SUB = 128
EROWS = 3200
E_PAD = EROWS * SUB
STAGE = 8

_MESH = plsc.VectorSubcoreMesh(core_axis_name="c", subcore_axis_name="s")


def _zero_rows(buf, nrows):
    zero = jnp.zeros((LANES,), jnp.float32)

    def body(i, _):
        for q in range(COLS // LANES):
            buf[i, pl.ds(q * LANES, LANES)] = zero
        return 0

    lax.fori_loop(0, nrows, body, 0, unroll=4)


def _spmm_sc(n_acc, mode, yt_off, x_rows):
    rows_per_tile = EROWS // NS
    chunks = rows_per_tile // STAGE
    if n_acc == 30000:
        passes, rpp, acc_rows, stripe, estripe = 2, 15000, 15040, 940, 188
    else:
        passes, rpp, acc_rows, stripe, estripe = 1, n_acc, n_acc, n_acc // NS, n_acc // NS // 5
    n_ep = stripe // estripe
    n_out = (passes - 1) * rpp + acc_rows

    scratch = [
        pltpu.VMEM_SHARED((acc_rows, COLS), jnp.float32),
    ] + ([pltpu.VMEM_SHARED((acc_rows, COLS), jnp.float32)]
         if mode == 3 else []) + [
        pltpu.VMEM((STAGE, SUB), jnp.int32),
        pltpu.VMEM((STAGE, SUB), jnp.int32),
        pltpu.VMEM((STAGE, SUB), jnp.int32),
        pltpu.VMEM((STAGE, SUB), jnp.float32),
        pltpu.VMEM((STAGE, SUB), jnp.float32),
    ] + [pltpu.VMEM((SUB, COLS), jnp.float32)] * 4 + (
        [] if mode == 3 else
        [pltpu.VMEM((estripe, COLS), jnp.float32),
         pltpu.VMEM((estripe, COLS), jnp.float32)]
    ) + [pltpu.SemaphoreType.DMA] * 8

    def body(*refs):
        acc2 = None
        if mode == 0:
            (x_hbm, gi_hbm, si_hbm, ev_hbm, out_hbm,
             acc, gi_v, si_v, sl_v, ev_v, em_v, gb0, gb1, gb2, gb3,
             obuf, ytbuf, *sems) = refs
            yt_hbm = None
        elif mode == 3:
            (x_hbm, gi_hbm, si_hbm, ev_hbm, out_hbm,
             acc, acc2, gi_v, si_v, sl_v, ev_v, em_v, gb0, gb1, gb2, gb3,
             *sems) = refs
            obuf = ytbuf = None
            yt_hbm = None
        else:
            (x_hbm, gi_hbm, si_hbm, ev_hbm, yt_hbm, out_hbm,
             acc, gi_v, si_v, sl_v, ev_v, em_v, gb0, gb1, gb2, gb3,
             obuf, ytbuf, *sems) = refs
        c = lax.axis_index("c")
        s = lax.axis_index("s")

        ring = 2 if mode == 3 else 4
        gbufs = (gb0, gb1, gb2, gb3)[:ring]
        sbufs = (gb2, gb3)
        gsems = sems[:4]
        ssems = sems[4:8]
        zebuf = gb0 if mode == 3 else obuf

        def scale(srcb, dstb, vref, j):
            def sgrp(g, _):
                v16 = vref[j, pl.ds(g * LANES, LANES)]
                base = g * LANES
                for l in range(LANES):
                    v = v16[l]
                    for q in range(COLS // LANES):
                        sl = pl.ds(q * LANES, LANES)
                        dstb[base + l, sl] = srcb[base + l, sl] * v
                return 0

            lax.fori_loop(0, SUB // LANES, sgrp, 0)

        erow0 = s * rows_per_tile

        for p in range(passes):
            _zero_rows(zebuf, estripe)
            for t in range(n_ep):
                pltpu.sync_copy(zebuf.at[pl.ds(0, estripe)],
                                acc.at[pl.ds(s * stripe + t * estripe,
                                             estripe)])
                if mode == 3:
                    pltpu.sync_copy(
                        zebuf.at[pl.ds(0, estripe)],
                        acc2.at[pl.ds(s * stripe + t * estripe,
                                      estripe)])
            plsc.subcore_barrier()

            def chunk(t, _):
                r0 = erow0 + t * STAGE
                pltpu.sync_copy(gi_hbm.at[pl.ds(r0, STAGE)], gi_v)
                pltpu.sync_copy(si_hbm.at[pl.ds(r0, STAGE)], si_v)
                pltpu.sync_copy(ev_hbm.at[pl.ds(r0, STAGE)], ev_v)
                if passes > 1:
                    lo = p * rpp
                    def xform(j, _):
                        for g in range(SUB // LANES):
                            sl = pl.ds(g * LANES, LANES)
                            si16 = si_v[j, sl]
                            v16 = ev_v[j, sl]
                            owned = (si16 >= lo) & (si16 < lo + rpp)
                            sl_v[j, sl] = jnp.where(
                                owned, si16 - lo, si16 & 8191)
                            em_v[j, sl] = jnp.where(owned, v16, 0.0)
                        return 0
                    lax.fori_loop(0, STAGE, xform, 0)
                    sref, vref = sl_v, em_v
                else:
                    sref, vref = si_v, ev_v
                gcp = [None] * STAGE
                scp = [None] * STAGE
                rcp = [None] * STAGE
                if mode == 3:
                    gcp[0] = pltpu.async_copy(
                        x_hbm.at[c].at[gi_v.at[0]], gbufs[0], gsems[0])
                    for j in range(STAGE):
                        gcp[j].wait()
                        rcp[j] = pltpu.async_copy(
                            gbufs[j % 2], acc2.at[sref.at[j]],
                            gsems[2 + (j % 2)], add=True)
                        if j + 1 < STAGE:
                            if j - 1 >= 0:
                                rcp[j - 1].wait()
                            gcp[j + 1] = pltpu.async_copy(
                                x_hbm.at[c].at[gi_v.at[j + 1]],
                                gbufs[(j + 1) % 2], gsems[(j + 1) % 2])
                        if j - 2 >= 0:
                            scp[j - 2].wait()
                        scale(gbufs[j % 2], sbufs[j % 2], vref, j)
                        scp[j] = pltpu.async_copy(
                            sbufs[j % 2], acc.at[sref.at[j]],
                            ssems[j % 2], add=True)
                    rcp[STAGE - 2].wait()
                    rcp[STAGE - 1].wait()
                    scp[STAGE - 2].wait()
                    scp[STAGE - 1].wait()
                    return 0
                ahead = ring // 2
                for j in range(ahead):
                    gcp[j] = pltpu.async_copy(
                        x_hbm.at[c].at[gi_v.at[j]], gbufs[j], gsems[j])
                for j in range(STAGE):
                    gcp[j].wait()
                    if j + ahead < STAGE:
                        if j + ahead - ring >= 0:
                            scp[j + ahead - ring].wait()
                        r = (j + ahead) % ring
                        gcp[j + ahead] = pltpu.async_copy(
                            x_hbm.at[c].at[gi_v.at[j + ahead]],
                            gbufs[r], gsems[r])
                    gbuf = gbufs[j % ring]
                    scale(gbuf, gbuf, vref, j)
                    scp[j] = pltpu.async_copy(
                        gbuf, acc.at[sref.at[j]], ssems[j % ring], add=True)
                for j in range(max(STAGE - ring, 0), STAGE):
                    scp[j].wait()
                return 0

            lax.fori_loop(0, chunks, chunk, 0)
            plsc.subcore_barrier()

            for t in range(n_ep):
                base = s * stripe + t * estripe
                gbase = p * rpp + base
                if mode == 0:
                    pltpu.sync_copy(acc.at[pl.ds(base, estripe)],
                                    out_hbm.at[c].at[pl.ds(gbase, estripe)])
                elif mode == 3:
                    pltpu.sync_copy(acc.at[pl.ds(base, estripe)],
                                    gb0.at[pl.ds(0, estripe)])
                    pltpu.sync_copy(acc2.at[pl.ds(base, estripe)],
                                    gb1.at[pl.ds(0, estripe)])

                    def drow(k, _):
                        for q in range(COLS // LANES):
                            sl = pl.ds(q * LANES, LANES)
                            cnt = jnp.maximum(gb1[k, sl], 1.0)
                            gb0[k, sl] = gb0[k, sl] / cnt
                        return 0

                    lax.fori_loop(0, estripe, drow, 0, unroll=4)
                    pltpu.sync_copy(gb0.at[pl.ds(0, estripe)],
                                    out_hbm.at[c].at[pl.ds(gbase, estripe)])
                else:
                    pltpu.sync_copy(acc.at[pl.ds(base, estripe)], obuf)
                    pltpu.sync_copy(
                        yt_hbm.at[c].at[pl.ds(yt_off + gbase, estripe)],
                        ytbuf)

                    def crow(k, _):
                        for q in range(COLS // LANES):
                            sl = pl.ds(q * LANES, LANES)
                            a = obuf[k, sl]
                            bv = ytbuf[k, sl]
                            obuf[k, sl] = bv + a if mode == 1 else bv - a
                        return 0

                    lax.fori_loop(0, estripe, crow, 0, unroll=4)
                    pltpu.sync_copy(obuf,
                                    out_hbm.at[c].at[pl.ds(gbase, estripe)])
            if p + 1 < passes:
                plsc.subcore_barrier()

    fn = pl.kernel(
        body,
        out_type=jax.ShapeDtypeStruct((NC, n_out, COLS), jnp.float32),
        mesh=_MESH,
        scratch_types=scratch,
        compiler_params=pltpu.CompilerParams(use_tc_tiling_on_sc=False),
    )
    return lambda *a: fn(*a)[:, :n_acc]


def _idl_c_kernel(left_ref, at_ref, c_ref):
    @pl.when(pl.program_id(0) == 0)
    def _():
        c_ref[...] = jnp.zeros_like(c_ref)

    c_ref[...] += lax.dot_general(
        left_ref[...], at_ref[...], (((0,), (0,)), ((), ())),
        preferred_element_type=jnp.float32)

    @pl.when(pl.program_id(0) == pl.num_programs(0) - 1)
    def _():
        c_ref[...] = c_ref[...] * IDL_BETA


def _idl_out_kernel(right_ref, c_ref, out_ref):
    out_ref[...] = lax.dot_general(
        right_ref[...], c_ref[...], (((0,), (0,)), ((), ())),
        preferred_element_type=jnp.float32)


def _idl_t(left_i, right_mat, at_b):
    ii, f = left_i.shape
    n = right_mat.shape[1]
    b = at_b.shape[1]
    ti = 2000
    c = pl.pallas_call(
        _idl_c_kernel,
        grid=(ii // ti,),
        in_specs=[pl.BlockSpec((ti, f), lambda i: (i, 0)),
                  pl.BlockSpec((ti, b), lambda i: (i, 0))],
        out_specs=pl.BlockSpec((f, b), lambda i: (0, 0)),
        out_shape=jax.ShapeDtypeStruct((f, b), jnp.float32),
    )(left_i, at_b)
    tn = 4096
    return pl.pallas_call(
        _idl_out_kernel,
        grid=(pl.cdiv(n, tn),),
        in_specs=[pl.BlockSpec((f, tn), lambda i: (0, i)),
                  pl.BlockSpec((f, b), lambda i: (0, 0))],
        out_specs=pl.BlockSpec((tn, b), lambda i: (i, 0)),
        out_shape=jax.ShapeDtypeStruct((n, b), jnp.float32),
    )(right_mat, c)


def kernel(batch_users, edge_rows, edge_cols, edge_vals, left_mat, right_mat):
    e2 = edge_rows.shape[0]
    e = e2 // 2
    n, f = left_mat.shape
    u_n = NUM_USER
    i_n = n - u_n
    b = batch_users.shape[0]
    pad = E_PAD - e

    eu = edge_rows[:e]
    ei = edge_cols[:e] - u_n
    ev = edge_vals[:e]
    zi = jnp.zeros((pad,), jnp.int32)
    eu_s = jnp.concatenate([eu, zi]).reshape(EROWS, SUB)
    ei_s = jnp.concatenate([ei, zi]).reshape(EROWS, SUB)
    ev_p = jnp.concatenate([ev, jnp.zeros((pad,), jnp.float32)]
                           ).reshape(EROWS, SUB)

    pos_tab = jnp.full((u_n,), b, jnp.int32).at[batch_users].set(
        jnp.arange(b, dtype=jnp.int32))
    rep = 64
    p_p = jnp.concatenate(
        [pos_tab[eu] + (b + 8) * (eu & (rep - 1)),
         jnp.full((pad,), b, jnp.int32)]).reshape(EROWS, SUB)
    x_id1 = jnp.concatenate(
        [jnp.eye(b, dtype=jnp.float32).reshape(b, NC, COLS).transpose(
            1, 0, 2), jnp.zeros((NC, 8, COLS), jnp.float32)], axis=1)
    x_id = jnp.tile(x_id1, (1, rep, 1))

    xs_at = _spmm_sc(i_n, 3, 0, b)(x_id, p_p, ei_s, ev_p)
    at_b = xs_at.transpose(1, 0, 2).reshape(i_n, b)

    yt0 = _idl_t(left_mat[u_n:], right_mat, at_b)
    yt0_s = yt0.reshape(n, NC, COLS).transpose(1, 0, 2)

    spmm_ui = _spmm_sc(u_n, 0, 0, i_n)
    s1 = spmm_ui(xs_at, ei_s, eu_s, ev_p)
    yt_i = _spmm_sc(i_n, 1, u_n, u_n)(s1, eu_s, ei_s, ev_p, yt0_s)
    q1 = _spmm_sc(i_n, 0, 0, n)(yt0_s, eu_s, ei_s, ev_p)
    out_u = _spmm_sc(u_n, 2, 0, i_n)(q1, ei_s, eu_s, ev_p, yt0_s)
    r1 = spmm_ui(yt_i, ei_s, eu_s, ev_p)
    out_i = _spmm_sc(i_n, 2, 0, u_n)(r1, eu_s, ei_s, ev_p, yt_i)

    out_u_b = out_u.transpose(0, 2, 1).reshape(b, u_n)
    out_i_b = out_i.transpose(0, 2, 1).reshape(b, i_n)
    return jnp.concatenate([out_u_b, out_i_b], axis=1)
